# Initial kernel scaffold; baseline (speedup 1.0000x reference)
import jax
import jax.numpy as jnp
from jax.experimental import pallas as pl


def kernel(tokens, emb_table, W1, b1, W2, b2, W3, b3, evall):
    raise NotImplementedError("write your pallas kernel here")


# speedup vs baseline: 8.8281x; 8.8281x over previous
import functools

import jax
import jax.numpy as jnp
from jax import lax
from jax.experimental import pallas as pl
from jax.experimental.pallas import tpu as pltpu
from jax.experimental.pallas import tpu_sc as plsc

B = 1024
L = 200
EMB = 128
NCLS = 100

NC = 2
NS = 16
NW = NC * NS
SPW = B // NW
CH = 100
CPW = SPW * (L // CH)
NCOL = EMB // 16


def _pooled_body(tokens_hbm, table_hbm, out_hbm, idx_v, buf0, buf1, outb,
                 sem0, sem1):
    wid = lax.axis_index("s") * NC + lax.axis_index("c")
    cbase = wid * CPW
    pltpu.sync_copy(tokens_hbm.at[pl.ds(cbase, CPW)], idx_v)

    bufs = (buf0, buf1)
    sems = (sem0, sem1)
    pltpu.async_copy(table_hbm.at[idx_v.at[0]], buf0, sem0)
    pltpu.async_copy(table_hbm.at[idx_v.at[1]], buf1, sem1)

    def reduce_rows(buf, acc):
        def rbody(r, a):
            return tuple(a[c] + buf[r, pl.ds(c * 16, 16)] for c in range(NCOL))
        return lax.fori_loop(0, CH, rbody, acc)

    zeros8 = tuple(jnp.zeros((16,), jnp.float32) for _ in range(NCOL))

    def sent_body(s, carry):
        acc = zeros8
        for h in range(2):
            g = 2 * s + h
            buf, sem = bufs[h], sems[h]
            pltpu.make_async_copy(table_hbm.at[idx_v.at[0]], buf, sem).wait()
            acc = reduce_rows(buf, acc)

            @pl.when(g + 2 < CPW)
            def _():
                pltpu.async_copy(table_hbm.at[idx_v.at[g + 2]], buf, sem)
        for c in range(NCOL):
            outb[s, pl.ds(c * 16, 16)] = acc[c]
        return carry

    lax.fori_loop(0, SPW, sent_body, 0)
    pltpu.sync_copy(outb, out_hbm.at[pl.ds(wid * SPW, SPW)])


@jax.jit
def _pooled_sum(tokens2d, emb_table):
    mesh = plsc.VectorSubcoreMesh(core_axis_name="c", subcore_axis_name="s")
    f = pl.kernel(
        _pooled_body,
        out_type=jax.ShapeDtypeStruct((B, EMB), jnp.float32),
        mesh=mesh,
        scratch_types=[
            pltpu.VMEM((CPW, CH), jnp.int32),
            pltpu.VMEM((CH, EMB), jnp.float32),
            pltpu.VMEM((CH, EMB), jnp.float32),
            pltpu.VMEM((SPW, EMB), jnp.float32),
            pltpu.SemaphoreType.DMA,
            pltpu.SemaphoreType.DMA,
        ],
    )
    return f(tokens2d, emb_table)


def _mlp_body(x_ref, w1_ref, b1_ref, w2_ref, b2_ref, w3_ref, b3_ref, o_ref):
    x = x_ref[...] * (1.0 / L)
    h = jnp.dot(x, w1_ref[...], preferred_element_type=jnp.float32)
    h = jnp.maximum(h + b1_ref[...], 0.0)
    h = jnp.dot(h, w2_ref[...], preferred_element_type=jnp.float32)
    h = jnp.maximum(h + b2_ref[...], 0.0)
    o_ref[...] = jnp.dot(h, w3_ref[...], preferred_element_type=jnp.float32) \
        + b3_ref[...]


_mlp = pl.pallas_call(
    _mlp_body,
    out_shape=jax.ShapeDtypeStruct((B, 128), jnp.float32),
)


def kernel(tokens, emb_table, W1, b1, W2, b2, W3, b3, evall=True):
    del evall
    tok2 = tokens.reshape(B * L // CH, CH)
    pooled = _pooled_sum(tok2, emb_table)

    f32 = jnp.float32
    HID = W1.shape[1]
    W1p = jnp.zeros((EMB, 128), f32).at[:, :HID].set(W1)
    b1p = jnp.zeros((1, 128), f32).at[0, :HID].set(b1)
    W2p = jnp.zeros((128, 128), f32).at[:HID, :HID].set(W2)
    b2p = jnp.zeros((1, 128), f32).at[0, :HID].set(b2)
    W3p = jnp.zeros((128, 128), f32).at[:HID, :NCLS].set(W3)
    b3p = jnp.zeros((1, 128), f32).at[0, :NCLS].set(b3)

    logits = _mlp(pooled, W1p, b1p, W2p, b2p, W3p, b3p)
    return logits[:, :NCLS]


# speedup vs baseline: 11.6111x; 1.3153x over previous
import jax
import jax.numpy as jnp
from jax import lax
from jax.experimental import pallas as pl
from jax.experimental.pallas import tpu as pltpu
from jax.experimental.pallas import tpu_sc as plsc

B = 1024
L = 200
EMB = 128
NCLS = 100

NC = 2
NS = 16
NW = NC * NS
SPW = B // NW
CH = 100
CPS = L // CH
CPW = SPW * CPS
NCOL = EMB // 16
NBUF = 4
NG = CPW // NBUF


def _pooled_body(tokens_hbm, table_hbm, out_hbm, idx_v, bufs, outb, sems):
    wid = lax.axis_index("s") * NC + lax.axis_index("c")
    cbase = wid * CPW
    pltpu.sync_copy(tokens_hbm.at[pl.ds(cbase, CPW)], idx_v)

    for j in range(NBUF):
        pltpu.async_copy(table_hbm.at[idx_v.at[j]], bufs[j], sems[j])

    def reduce_rows(buf, acc):
        def rbody(r, a):
            return tuple(a[c] + buf[r, pl.ds(c * 16, 16)] for c in range(NCOL))
        return lax.fori_loop(0, CH, rbody, acc)

    zeros8 = tuple(jnp.zeros((16,), jnp.float32) for _ in range(NCOL))

    def group_body(gi, carry):
        g0 = gi * NBUF
        accs = []
        acc = zeros8
        for j in range(NBUF):
            pltpu.make_async_copy(table_hbm.at[idx_v.at[0]], bufs[j],
                                  sems[j]).wait()
            acc = reduce_rows(bufs[j], acc)
            if j % CPS == CPS - 1:
                accs.append(acc)
                acc = zeros8

            @pl.when(g0 + j + NBUF < CPW)
            def _():
                pltpu.async_copy(table_hbm.at[idx_v.at[g0 + j + NBUF]],
                                 bufs[j], sems[j])
        s0 = gi * (NBUF // CPS)
        for t, a in enumerate(accs):
            for c in range(NCOL):
                outb[s0 + t, pl.ds(c * 16, 16)] = a[c]
        return carry

    lax.fori_loop(0, NG, group_body, 0)
    pltpu.sync_copy(outb, out_hbm.at[pl.ds(wid * SPW, SPW)])


def _pooled_sum(tokens2d, emb_table):
    mesh = plsc.VectorSubcoreMesh(core_axis_name="c", subcore_axis_name="s")
    f = pl.kernel(
        _pooled_body,
        out_type=jax.ShapeDtypeStruct((B, EMB), jnp.float32),
        mesh=mesh,
        scratch_types=[
            pltpu.VMEM((CPW, CH), jnp.int32),
            tuple(pltpu.VMEM((CH, EMB), jnp.float32) for _ in range(NBUF)),
            pltpu.VMEM((SPW, EMB), jnp.float32),
            tuple(pltpu.SemaphoreType.DMA for _ in range(NBUF)),
        ],
    )
    return f(tokens2d, emb_table)


def _mlp_body(x_ref, w1_ref, b1_ref, w2_ref, b2_ref, w3_ref, b3_ref, o_ref):
    x = x_ref[...] * (1.0 / L)
    h = jnp.dot(x, w1_ref[...], preferred_element_type=jnp.float32)
    h = jnp.maximum(h + b1_ref[...], 0.0)
    h = jnp.dot(h, w2_ref[...], preferred_element_type=jnp.float32)
    h = jnp.maximum(h + b2_ref[...], 0.0)
    o_ref[...] = jnp.dot(h, w3_ref[...], preferred_element_type=jnp.float32) \
        + b3_ref[...]


def kernel(tokens, emb_table, W1, b1, W2, b2, W3, b3, evall=True):
    del evall
    tok2 = tokens.reshape(B * L // CH, CH)
    pooled = _pooled_sum(tok2, emb_table)

    hid = W1.shape[1]
    mlp = pl.pallas_call(
        _mlp_body,
        out_shape=jax.ShapeDtypeStruct((B, NCLS), jnp.float32),
    )
    return mlp(pooled, W1, b1.reshape(1, hid), W2, b2.reshape(1, hid),
               W3, b3.reshape(1, NCLS))
